# baseline (speedup 1.0000x reference)
import functools

import jax
import jax.numpy as jnp
from jax import lax
from jax.experimental import pallas as pl
from jax.experimental.pallas import tpu as pltpu
from jax.experimental.pallas import tpu_sc as plsc

_NUM_ITEMS = 20000
_NUM_USERS = 30000
_N = _NUM_ITEMS + _NUM_USERS
_E = 800000
_D = 64
_NL = 3
_B = 2048

_NC = 2
_NS = 16
_L = 16

_NHALF = _N // 2
_PAD = 120
_ACC_ROWS = _NHALF + _PAD
_DUMMY_MASK = 63

_CHUNK = 128
_NCHUNKS = _E // _CHUNK
_NK = _NCHUNKS // _NS
_NLEFT = _NCHUNKS - _NK * _NS

_ZROWS = 80
_ZCHUNKS = _ACC_ROWS // _ZROWS
_WB_ROWS = 200
_WB_CHUNKS = _NHALF // _WB_ROWS


def _prop_body(emb, col1, row1, vals1, out,
               colv0, rowv0, valsv0, sidx0, rows0,
               colv1, rowv1, valsv1, sidx1, rows1,
               acc, gsem0, gsem1, ssem0, ssem1, isem0, isem1):
    c = lax.axis_index("c")
    s = lax.axis_index("s")
    base = c * _NHALF

    colv = (colv0, colv1)
    rowv = (rowv0, rowv1)
    valsv = (valsv0, valsv1)
    sidx = (sidx0, sidx1)
    rows = (rows0, rows1)
    gsem = (gsem0, gsem1)
    ssem = (ssem0, ssem1)
    isem = (isem0, isem1)

    def _zrow(r, _):
        for j in range(_D // _L):
            rows0[r, pl.ds(j * _L, _L)] = jnp.zeros((_L,), jnp.float32)
        return _
    lax.fori_loop(0, _ZROWS, _zrow, None)

    def _zacc(k, _):
        cid = s + _NS * k
        @pl.when(cid < _ZCHUNKS)
        def _():
            pltpu.sync_copy(rows0.at[pl.ds(0, _ZROWS)],
                            acc.at[pl.ds(cid * _ZROWS, _ZROWS)])
        return _
    lax.fori_loop(0, (_ZCHUNKS + _NS - 1) // _NS, _zacc, None)
    plsc.subcore_barrier()

    def _ebase(k):
        return (k * _NS + s) * _CHUNK

    def _issue_idx(k, p):
        eb = _ebase(k)
        pltpu.async_copy(col1.at[pl.ds(eb, _CHUNK)], colv[p], isem[p])
        pltpu.async_copy(row1.at[pl.ds(eb, _CHUNK)], rowv[p], isem[p])
        pltpu.async_copy(vals1.at[pl.ds(eb, _CHUNK)], valsv[p], isem[p])

    def _wait_idx(k, p):
        eb = _ebase(k)
        pltpu.make_async_copy(col1.at[pl.ds(eb, _CHUNK)], colv[p],
                              isem[p]).wait()
        pltpu.make_async_copy(row1.at[pl.ds(eb, _CHUNK)], rowv[p],
                              isem[p]).wait()
        pltpu.make_async_copy(vals1.at[pl.ds(eb, _CHUNK)], valsv[p],
                              isem[p]).wait()

    def _issue_gather(p):
        pltpu.async_copy(emb.at[colv[p]], rows[p], gsem[p])

    def _wait_gather(p):
        pltpu.make_async_copy(emb.at[colv[p]], rows[p], gsem[p]).wait()

    def _issue_scatter(p):
        pltpu.async_copy(rows[p], acc.at[sidx[p]], ssem[p], add=True)

    def _wait_scatter(p):
        pltpu.make_async_copy(rows[p], acc.at[sidx[p]], ssem[p]).wait()

    def _compute(p):
        @plsc.parallel_loop(0, _CHUNK // _L, unroll=4)
        def _group(g):
            o = g * _L
            r16 = rowv[p][pl.ds(o, _L)]
            local = r16 - base
            okm = (local >= 0) & (local < _NHALF)
            dum = _NHALF + (r16 & _DUMMY_MASK)
            sidx[p][pl.ds(o, _L)] = jnp.where(okm, local, dum)

            v16 = valsv[p][pl.ds(o, _L)]
            dn = lax.GatherDimensionNumbers(
                offset_dims=(), collapsed_slice_dims=(0,),
                start_index_map=(0,))
            for l in range(_L):
                idx = jnp.full((_L, 1), l, jnp.int32)
                splat = lax.gather(
                    v16, idx, dn, slice_sizes=(1,),
                    mode=lax.GatherScatterMode.PROMISE_IN_BOUNDS)
                for q in range(_D // _L):
                    seg = rows[p][o + l, pl.ds(q * _L, _L)]
                    rows[p][o + l, pl.ds(q * _L, _L)] = seg * splat

    def _sub(k, p, first, last):
        _wait_gather(p)
        if not first:
            _wait_scatter(1 - p)
        if not last:
            _wait_idx(k + 1, 1 - p)
            _issue_gather(1 - p)
        _compute(p)
        _issue_scatter(p)
        if not last:
            @pl.when(k + 2 < _NK)
            def _():
                _issue_idx(k + 2, p)

    pltpu.sync_copy(col1.at[pl.ds(_ebase(0), _CHUNK)], colv[0])
    pltpu.sync_copy(row1.at[pl.ds(_ebase(0), _CHUNK)], rowv[0])
    pltpu.sync_copy(vals1.at[pl.ds(_ebase(0), _CHUNK)], valsv[0])
    _issue_gather(0)
    _issue_idx(1, 1)

    def _dbody(t, _):
        k = 2 * t + 1
        _sub(k, 1, False, False)
        _sub(k + 1, 0, False, False)
        return _

    _sub(0, 0, True, False)
    lax.fori_loop(0, (_NK - 2) // 2, _dbody, None)
    _sub(_NK - 1, 1, False, True)
    _wait_scatter(1)

    @pl.when(s < _NLEFT)
    def _():
        eb = (_NK * _NS + s) * _CHUNK
        pltpu.sync_copy(col1.at[pl.ds(eb, _CHUNK)], colv[0])
        pltpu.sync_copy(row1.at[pl.ds(eb, _CHUNK)], rowv[0])
        pltpu.sync_copy(vals1.at[pl.ds(eb, _CHUNK)], valsv[0])
        pltpu.async_copy(emb.at[colv[0]], rows[0], gsem[0]).wait()
        _compute(0)
        pltpu.async_copy(rows[0], acc.at[sidx[0]], ssem[0], add=True).wait()

    plsc.subcore_barrier()

    def _wb(k, _):
        cid = s + _NS * k
        @pl.when(cid < _WB_CHUNKS)
        def _():
            pltpu.sync_copy(
                acc.at[pl.ds(cid * _WB_ROWS, _WB_ROWS)],
                out.at[pl.ds(base + cid * _WB_ROWS, _WB_ROWS)])
        return _
    lax.fori_loop(0, (_WB_CHUNKS + _NS - 1) // _NS, _wb, None)


_prop = functools.partial(
    pl.kernel,
    out_type=jax.ShapeDtypeStruct((_N, _D), jnp.float32),
    compiler_params=pltpu.CompilerParams(use_tc_tiling_on_sc=False),
    mesh=plsc.VectorSubcoreMesh(core_axis_name="c", subcore_axis_name="s",
                                num_cores=_NC, num_subcores=_NS),
    scratch_types=(
        [
            pltpu.VMEM((_CHUNK,), jnp.int32),
            pltpu.VMEM((_CHUNK,), jnp.int32),
            pltpu.VMEM((_CHUNK,), jnp.float32),
            pltpu.VMEM((_CHUNK,), jnp.int32),
            pltpu.VMEM((_CHUNK, _D), jnp.float32),
        ] * 2
        + [pltpu.VMEM_SHARED((_ACC_ROWS, _D), jnp.float32)]
        + [pltpu.SemaphoreType.DMA] * 6
    ),
)(_prop_body)


_GB = 64
_GCHUNKS = 3 * _B // _GB


def _gather_body(e0, e1, e2, e3, items, pos, neg, g0, g1, g2, g3,
                 idxv, rowbuf, sem):
    c = lax.axis_index("c")
    s = lax.axis_index("s")
    w = s * _NC + c

    def _chunk(k, _):
        cid = w + _NC * _NS * k
        a = cid // (_B // _GB)
        q = cid % (_B // _GB)

        @pl.when(a == 0)
        def _():
            pltpu.sync_copy(items.at[pl.ds(q * _GB, _GB)], idxv)
        @pl.when(a == 1)
        def _():
            pltpu.sync_copy(pos.at[pl.ds(q * _GB, _GB)], idxv)
        @pl.when(a == 2)
        def _():
            pltpu.sync_copy(neg.at[pl.ds(q * _GB, _GB)], idxv)

        off = jnp.where(a == 0, 0, _NUM_ITEMS).astype(jnp.int32)
        for g in range(_GB // _L):
            idxv[pl.ds(g * _L, _L)] = idxv[pl.ds(g * _L, _L)] + off

        for tbl, outt in ((e0, g0), (e1, g1), (e2, g2), (e3, g3)):
            pltpu.async_copy(tbl.at[idxv], rowbuf, sem).wait()
            pltpu.sync_copy(rowbuf, outt.at[pl.ds(cid * _GB, _GB)])
        return _
    lax.fori_loop(0, _GCHUNKS // (_NC * _NS), _chunk, None)


_gather = functools.partial(
    pl.kernel,
    out_type=(jax.ShapeDtypeStruct((3 * _B, _D), jnp.float32),) * 4,
    compiler_params=pltpu.CompilerParams(use_tc_tiling_on_sc=False),
    mesh=plsc.VectorSubcoreMesh(core_axis_name="c", subcore_axis_name="s",
                                num_cores=_NC, num_subcores=_NS),
    scratch_types=[
        pltpu.VMEM((_GB,), jnp.int32),
        pltpu.VMEM((_GB, _D), jnp.float32),
        pltpu.SemaphoreType.DMA,
    ],
)(_gather_body)


def _loss_body(g0, g1, g2, g3, loss_ref, reg_ref):
    light = (g0[...] + g1[...] + g2[...] + g3[...]) * 0.25
    items_emb = light[0:_B]
    pos_emb = light[_B:2 * _B]
    neg_emb = light[2 * _B:3 * _B]
    pos_scores = jnp.sum(items_emb * pos_emb, axis=1)
    neg_scores = jnp.sum(items_emb * neg_emb, axis=1)
    loss_ref[0] = jnp.mean(jax.nn.softplus(neg_scores - pos_scores))
    reg_ref[0] = 0.5 * jnp.sum(g0[...] ** 2) / float(_B)


def _loss_stage(g0, g1, g2, g3):
    loss, reg = pl.pallas_call(
        _loss_body,
        out_shape=(
            jax.ShapeDtypeStruct((1,), jnp.float32),
            jax.ShapeDtypeStruct((1,), jnp.float32),
        ),
        in_specs=[pl.BlockSpec(memory_space=pltpu.VMEM)] * 4,
        out_specs=(
            pl.BlockSpec(memory_space=pltpu.SMEM),
            pl.BlockSpec(memory_space=pltpu.SMEM),
        ),
    )(g0, g1, g2, g3)
    return loss[0], reg[0]


def kernel(item_table, user_table, edge_vals, edge_index, items, pos, neg):
    e0 = jnp.concatenate([item_table, user_table], axis=0)
    row1 = edge_index[0]
    col1 = edge_index[1]

    e1 = _prop(e0, col1, row1, edge_vals)
    e2 = _prop(e1, col1, row1, edge_vals)
    e3 = _prop(e2, col1, row1, edge_vals)

    g0, g1, g2, g3 = _gather(e0, e1, e2, e3, items, pos, neg)
    loss, reg = _loss_stage(g0, g1, g2, g3)
    return (loss, reg)


# speedup vs baseline: 1.1165x; 1.1165x over previous
import functools

import jax
import jax.numpy as jnp
from jax import lax
from jax.experimental import pallas as pl
from jax.experimental.pallas import tpu as pltpu
from jax.experimental.pallas import tpu_sc as plsc

_NUM_ITEMS = 20000
_NUM_USERS = 30000
_N = _NUM_ITEMS + _NUM_USERS
_E = 800000
_D = 64
_DH = _D // 2
_NL = 3
_B = 2048

_NC = 2
_NS = 16
_L = 16

_CHUNK = 128
_NCHUNKS = _E // _CHUNK
_NK = _NCHUNKS // _NS
_NLEFT = _NCHUNKS - _NK * _NS

_ZROWS = 125
_ZCHUNKS = _N // _ZROWS
_WB_ROWS = 200
_WB_CHUNKS = _N // _WB_ROWS


def _prop_body(emb, col1, row1, vals1, out,
               colv0, valsv0, sidx0, rows0,
               colv1, valsv1, sidx1, rows1,
               acc, gsem0, gsem1, ssem0, ssem1, isem0, isem1):
    c = lax.axis_index("c")
    s = lax.axis_index("s")
    hoff = c * _N

    colv = (colv0, colv1)
    valsv = (valsv0, valsv1)
    sidx = (sidx0, sidx1)
    rows = (rows0, rows1)
    gsem = (gsem0, gsem1)
    ssem = (ssem0, ssem1)
    isem = (isem0, isem1)

    def _zrow(r, _):
        for j in range(_DH // _L):
            rows0[r, pl.ds(j * _L, _L)] = jnp.zeros((_L,), jnp.float32)
        return _
    lax.fori_loop(0, _ZROWS, _zrow, None)

    def _zacc(k, _):
        cid = s + _NS * k
        pltpu.sync_copy(rows0.at[pl.ds(0, _ZROWS)],
                        acc.at[pl.ds(cid * _ZROWS, _ZROWS)])
        return _
    lax.fori_loop(0, _ZCHUNKS // _NS, _zacc, None)
    plsc.subcore_barrier()

    def _ebase(k):
        return (k * _NS + s) * _CHUNK

    def _issue_idx(k, p):
        eb = _ebase(k)
        pltpu.async_copy(col1.at[pl.ds(eb, _CHUNK)], colv[p], isem[p])
        pltpu.async_copy(row1.at[pl.ds(eb, _CHUNK)], sidx[p], isem[p])
        pltpu.async_copy(vals1.at[pl.ds(eb, _CHUNK)], valsv[p], isem[p])

    def _wait_idx(k, p):
        eb = _ebase(k)
        pltpu.make_async_copy(col1.at[pl.ds(eb, _CHUNK)], colv[p],
                              isem[p]).wait()
        pltpu.make_async_copy(row1.at[pl.ds(eb, _CHUNK)], sidx[p],
                              isem[p]).wait()
        pltpu.make_async_copy(vals1.at[pl.ds(eb, _CHUNK)], valsv[p],
                              isem[p]).wait()
        @plsc.parallel_loop(0, _CHUNK // _L, unroll=2)
        def _adj(g):
            o = g * _L
            colv[p][pl.ds(o, _L)] = colv[p][pl.ds(o, _L)] + hoff

    def _issue_gather(p):
        pltpu.async_copy(emb.at[colv[p]], rows[p], gsem[p])

    def _wait_gather(p):
        pltpu.make_async_copy(emb.at[colv[p]], rows[p], gsem[p]).wait()

    def _issue_scatter(p):
        pltpu.async_copy(rows[p], acc.at[sidx[p]], ssem[p], add=True)

    def _wait_scatter(p):
        pltpu.make_async_copy(rows[p], acc.at[sidx[p]], ssem[p]).wait()

    def _compute(p):
        @plsc.parallel_loop(0, _CHUNK // _L, unroll=2)
        def _group(g):
            o = g * _L
            v16 = valsv[p][pl.ds(o, _L)]
            dn = lax.GatherDimensionNumbers(
                offset_dims=(), collapsed_slice_dims=(0,),
                start_index_map=(0,))
            for l in range(_L):
                idx = jnp.full((_L, 1), l, jnp.int32)
                splat = lax.gather(
                    v16, idx, dn, slice_sizes=(1,),
                    mode=lax.GatherScatterMode.PROMISE_IN_BOUNDS)
                for q in range(_DH // _L):
                    seg = rows[p][o + l, pl.ds(q * _L, _L)]
                    rows[p][o + l, pl.ds(q * _L, _L)] = seg * splat

    def _sub(k, p, first, last):
        _wait_gather(p)
        if not first:
            _wait_scatter(1 - p)
        if not last:
            _wait_idx(k + 1, 1 - p)
            _issue_gather(1 - p)
        _compute(p)
        _issue_scatter(p)
        if not last:
            @pl.when(k + 2 < _NK)
            def _():
                _issue_idx(k + 2, p)

    pltpu.sync_copy(col1.at[pl.ds(_ebase(0), _CHUNK)], colv[0])
    pltpu.sync_copy(row1.at[pl.ds(_ebase(0), _CHUNK)], sidx[0])
    pltpu.sync_copy(vals1.at[pl.ds(_ebase(0), _CHUNK)], valsv[0])

    @plsc.parallel_loop(0, _CHUNK // _L, unroll=2)
    def _adj0(g):
        o = g * _L
        colv[0][pl.ds(o, _L)] = colv[0][pl.ds(o, _L)] + hoff

    _issue_gather(0)
    _issue_idx(1, 1)

    def _dbody(t, _):
        k = 2 * t + 1
        _sub(k, 1, False, False)
        _sub(k + 1, 0, False, False)
        return _

    _sub(0, 0, True, False)
    lax.fori_loop(0, (_NK - 2) // 2, _dbody, None)
    _sub(_NK - 1, 1, False, True)
    _wait_scatter(1)

    @pl.when(s < _NLEFT)
    def _():
        eb = (_NK * _NS + s) * _CHUNK
        pltpu.sync_copy(col1.at[pl.ds(eb, _CHUNK)], colv[0])
        pltpu.sync_copy(row1.at[pl.ds(eb, _CHUNK)], sidx[0])
        pltpu.sync_copy(vals1.at[pl.ds(eb, _CHUNK)], valsv[0])

        @plsc.parallel_loop(0, _CHUNK // _L, unroll=2)
        def _adjl(g):
            o = g * _L
            colv[0][pl.ds(o, _L)] = colv[0][pl.ds(o, _L)] + hoff

        pltpu.async_copy(emb.at[colv[0]], rows[0], gsem[0]).wait()
        _compute(0)
        pltpu.async_copy(rows[0], acc.at[sidx[0]], ssem[0], add=True).wait()

    plsc.subcore_barrier()

    def _wb(k, _):
        cid = s + _NS * k
        @pl.when(cid < _WB_CHUNKS)
        def _():
            pltpu.sync_copy(
                acc.at[pl.ds(cid * _WB_ROWS, _WB_ROWS)],
                out.at[pl.ds(hoff + cid * _WB_ROWS, _WB_ROWS)])
        return _
    lax.fori_loop(0, (_WB_CHUNKS + _NS - 1) // _NS, _wb, None)


_prop = functools.partial(
    pl.kernel,
    out_type=jax.ShapeDtypeStruct((_NC * _N, _DH), jnp.float32),
    compiler_params=pltpu.CompilerParams(use_tc_tiling_on_sc=False),
    mesh=plsc.VectorSubcoreMesh(core_axis_name="c", subcore_axis_name="s",
                                num_cores=_NC, num_subcores=_NS),
    scratch_types=(
        [
            pltpu.VMEM((_CHUNK,), jnp.int32),
            pltpu.VMEM((_CHUNK,), jnp.float32),
            pltpu.VMEM((_CHUNK,), jnp.int32),
            pltpu.VMEM((_CHUNK, _DH), jnp.float32),
        ] * 2
        + [pltpu.VMEM_SHARED((_N, _DH), jnp.float32)]
        + [pltpu.SemaphoreType.DMA] * 6
    ),
)(_prop_body)


_GB = 64
_GCHUNKS = 3 * _B // _GB


def _gather_body(e0, e1, e2, e3, items, pos, neg, g0, g1, g2, g3,
                 idxv, rowbuf, sem):
    c = lax.axis_index("c")
    s = lax.axis_index("s")
    w = s * _NC + c

    def _chunk(k, _):
        cid = w + _NC * _NS * k
        a = cid // (_B // _GB)
        q = cid % (_B // _GB)

        @pl.when(a == 0)
        def _():
            pltpu.sync_copy(items.at[pl.ds(q * _GB, _GB)], idxv)
        @pl.when(a == 1)
        def _():
            pltpu.sync_copy(pos.at[pl.ds(q * _GB, _GB)], idxv)
        @pl.when(a == 2)
        def _():
            pltpu.sync_copy(neg.at[pl.ds(q * _GB, _GB)], idxv)

        off = jnp.where(a == 0, 0, _NUM_ITEMS).astype(jnp.int32)
        for g in range(_GB // _L):
            idxv[pl.ds(g * _L, _L)] = idxv[pl.ds(g * _L, _L)] + off

        for h in range(_NC):
            if h:
                for g in range(_GB // _L):
                    idxv[pl.ds(g * _L, _L)] = idxv[pl.ds(g * _L, _L)] + _N
            for tbl, outt in ((e0, g0), (e1, g1), (e2, g2), (e3, g3)):
                pltpu.async_copy(tbl.at[idxv], rowbuf, sem).wait()
                pltpu.sync_copy(
                    rowbuf, outt.at[pl.ds(h * 3 * _B + cid * _GB, _GB)])
        return _
    lax.fori_loop(0, _GCHUNKS // (_NC * _NS), _chunk, None)


_gather = functools.partial(
    pl.kernel,
    out_type=(jax.ShapeDtypeStruct((_NC * 3 * _B, _DH), jnp.float32),) * 4,
    compiler_params=pltpu.CompilerParams(use_tc_tiling_on_sc=False),
    mesh=plsc.VectorSubcoreMesh(core_axis_name="c", subcore_axis_name="s",
                                num_cores=_NC, num_subcores=_NS),
    scratch_types=[
        pltpu.VMEM((_GB,), jnp.int32),
        pltpu.VMEM((_GB, _DH), jnp.float32),
        pltpu.SemaphoreType.DMA,
    ],
)(_gather_body)


def _loss_body(g0, g1, g2, g3, loss_ref, reg_ref):
    light = (g0[...] + g1[...] + g2[...] + g3[...]) * 0.25
    ps = jnp.zeros((_B,), jnp.float32)
    ns = jnp.zeros((_B,), jnp.float32)
    for h in range(_NC):
        o = h * 3 * _B
        items_emb = light[o:o + _B]
        pos_emb = light[o + _B:o + 2 * _B]
        neg_emb = light[o + 2 * _B:o + 3 * _B]
        ps = ps + jnp.sum(items_emb * pos_emb, axis=1)
        ns = ns + jnp.sum(items_emb * neg_emb, axis=1)
    loss_ref[0] = jnp.mean(jax.nn.softplus(ns - ps))
    reg_ref[0] = 0.5 * jnp.sum(g0[...] ** 2) / float(_B)


def _loss_stage(g0, g1, g2, g3):
    loss, reg = pl.pallas_call(
        _loss_body,
        out_shape=(
            jax.ShapeDtypeStruct((1,), jnp.float32),
            jax.ShapeDtypeStruct((1,), jnp.float32),
        ),
        in_specs=[pl.BlockSpec(memory_space=pltpu.VMEM)] * 4,
        out_specs=(
            pl.BlockSpec(memory_space=pltpu.SMEM),
            pl.BlockSpec(memory_space=pltpu.SMEM),
        ),
    )(g0, g1, g2, g3)
    return loss[0], reg[0]


def kernel(item_table, user_table, edge_vals, edge_index, items, pos, neg):
    e0 = jnp.concatenate([
        item_table[:, :_DH], user_table[:, :_DH],
        item_table[:, _DH:], user_table[:, _DH:],
    ], axis=0)
    row1 = edge_index[0]
    col1 = edge_index[1]

    e1 = _prop(e0, col1, row1, edge_vals)
    e2 = _prop(e1, col1, row1, edge_vals)
    e3 = _prop(e2, col1, row1, edge_vals)

    g0, g1, g2, g3 = _gather(e0, e1, e2, e3, items, pos, neg)
    loss, reg = _loss_stage(g0, g1, g2, g3)
    return (loss, reg)


# speedup vs baseline: 1.5372x; 1.3768x over previous
import functools

import jax
import jax.numpy as jnp
from jax import lax
from jax.experimental import pallas as pl
from jax.experimental.pallas import tpu as pltpu
from jax.experimental.pallas import tpu_sc as plsc

_NUM_ITEMS = 20000
_NUM_USERS = 30000
_N = _NUM_ITEMS + _NUM_USERS
_E = 800000
_D = 64
_DH = _D // 2
_NL = 3
_B = 2048

_NC = 2
_NS = 16
_L = 16

_CSUB = 128
_NSTR = 2
_CHUNK = _CSUB * _NSTR
_NCHUNKS = _E // _CHUNK
_NK = _NCHUNKS // _NS
_NLEFT = _NCHUNKS - _NK * _NS

_ZROWS = 125
_ZCHUNKS = _N // _ZROWS
_WB_ROWS = 200
_WB_CHUNKS = _N // _WB_ROWS


def _prop_body(emb, col1, row1, vals1, out,
               colv0, valsv0, sidx0, rows0,
               colv1, valsv1, sidx1, rows1,
               acc, gsem0, gsem1, ssem0, ssem1, isem0, isem1):
    c = lax.axis_index("c")
    s = lax.axis_index("s")
    hoff = c * _N

    colv = (colv0, colv1)
    valsv = (valsv0, valsv1)
    sidx = (sidx0, sidx1)
    rows = (rows0, rows1)
    gsem = (gsem0, gsem1)
    ssem = (ssem0, ssem1)
    isem = (isem0, isem1)

    def _zrow(r, _):
        for j in range(_DH // _L):
            rows0[r, pl.ds(j * _L, _L)] = jnp.zeros((_L,), jnp.float32)
        return _
    lax.fori_loop(0, _ZROWS, _zrow, None)

    def _zacc(k, _):
        cid = s + _NS * k
        pltpu.sync_copy(rows0.at[pl.ds(0, _ZROWS)],
                        acc.at[pl.ds(cid * _ZROWS, _ZROWS)])
        return _
    lax.fori_loop(0, _ZCHUNKS // _NS, _zacc, None)
    plsc.subcore_barrier()

    def _ebase(k):
        return (k * _NS + s) * _CHUNK

    def _issue_idx(k, p):
        eb = _ebase(k)
        pltpu.async_copy(col1.at[pl.ds(eb, _CHUNK)], colv[p], isem[p])
        for j in range(_NSTR):
            pltpu.async_copy(row1.at[pl.ds(eb + j * _CSUB, _CSUB)],
                             sidx[p].at[j], isem[p])
        pltpu.async_copy(vals1.at[pl.ds(eb, _CHUNK)], valsv[p], isem[p])

    def _wait_idx(k, p):
        eb = _ebase(k)
        pltpu.make_async_copy(col1.at[pl.ds(eb, _CHUNK)], colv[p],
                              isem[p]).wait()
        for j in range(_NSTR):
            pltpu.make_async_copy(row1.at[pl.ds(eb + j * _CSUB, _CSUB)],
                                  sidx[p].at[j], isem[p]).wait()
        pltpu.make_async_copy(vals1.at[pl.ds(eb, _CHUNK)], valsv[p],
                              isem[p]).wait()
        @plsc.parallel_loop(0, _CHUNK // _L, unroll=2)
        def _adj(g):
            o = g * _L
            colv[p][pl.ds(o, _L)] = colv[p][pl.ds(o, _L)] + hoff

    def _issue_gather(p):
        for j in range(_NSTR):
            pltpu.async_copy(emb.at[colv[p].at[pl.ds(j * _CSUB, _CSUB)]],
                             rows[p].at[pl.ds(j * _CSUB, _CSUB)], gsem[p])

    def _wait_gather(p):
        for j in range(_NSTR):
            pltpu.make_async_copy(
                emb.at[colv[p].at[pl.ds(j * _CSUB, _CSUB)]],
                rows[p].at[pl.ds(j * _CSUB, _CSUB)], gsem[p]).wait()

    def _issue_scatter(p):
        for j in range(_NSTR):
            pltpu.async_copy(rows[p].at[pl.ds(j * _CSUB, _CSUB)],
                             acc.at[sidx[p].at[j]], ssem[p], add=True)

    def _wait_scatter(p):
        for j in range(_NSTR):
            pltpu.make_async_copy(rows[p].at[pl.ds(j * _CSUB, _CSUB)],
                                  acc.at[sidx[p].at[j]], ssem[p]).wait()

    def _compute(p):
        @plsc.parallel_loop(0, _CHUNK // _L, unroll=2)
        def _group(g):
            o = g * _L
            v16 = valsv[p][pl.ds(o, _L)]
            dn = lax.GatherDimensionNumbers(
                offset_dims=(), collapsed_slice_dims=(0,),
                start_index_map=(0,))
            for l in range(_L):
                idx = jnp.full((_L, 1), l, jnp.int32)
                splat = lax.gather(
                    v16, idx, dn, slice_sizes=(1,),
                    mode=lax.GatherScatterMode.PROMISE_IN_BOUNDS)
                for q in range(_DH // _L):
                    seg = rows[p][o + l, pl.ds(q * _L, _L)]
                    rows[p][o + l, pl.ds(q * _L, _L)] = seg * splat

    def _sub(k, p, first, last):
        _wait_gather(p)
        if not first:
            _wait_scatter(1 - p)
        if not last:
            _wait_idx(k + 1, 1 - p)
            _issue_gather(1 - p)
        _compute(p)
        _issue_scatter(p)
        if not last:
            @pl.when(k + 2 < _NK)
            def _():
                _issue_idx(k + 2, p)

    pltpu.sync_copy(col1.at[pl.ds(_ebase(0), _CHUNK)], colv[0])
    for j in range(_NSTR):
        pltpu.sync_copy(row1.at[pl.ds(_ebase(0) + j * _CSUB, _CSUB)],
                        sidx[0].at[j])
    pltpu.sync_copy(vals1.at[pl.ds(_ebase(0), _CHUNK)], valsv[0])

    @plsc.parallel_loop(0, _CHUNK // _L, unroll=2)
    def _adj0(g):
        o = g * _L
        colv[0][pl.ds(o, _L)] = colv[0][pl.ds(o, _L)] + hoff

    _issue_gather(0)
    _issue_idx(1, 1)

    def _dbody(t, _):
        k = 2 * t + 1
        _sub(k, 1, False, False)
        _sub(k + 1, 0, False, False)
        return _

    _sub(0, 0, True, False)
    lax.fori_loop(0, (_NK - 3) // 2, _dbody, None)
    _sub(_NK - 2, 1, False, False)
    _sub(_NK - 1, 0, False, True)
    _wait_scatter(0)

    @pl.when(s < _NLEFT)
    def _():
        eb = (_NK * _NS + s) * _CHUNK
        pltpu.sync_copy(col1.at[pl.ds(eb, _CHUNK)], colv[0])
        for j in range(_NSTR):
            pltpu.sync_copy(row1.at[pl.ds(eb + j * _CSUB, _CSUB)],
                            sidx[0].at[j])
        pltpu.sync_copy(vals1.at[pl.ds(eb, _CHUNK)], valsv[0])

        @plsc.parallel_loop(0, _CHUNK // _L, unroll=2)
        def _adjl(g):
            o = g * _L
            colv[0][pl.ds(o, _L)] = colv[0][pl.ds(o, _L)] + hoff

        _issue_gather(0)
        _wait_gather(0)
        _compute(0)
        _issue_scatter(0)
        _wait_scatter(0)

    plsc.subcore_barrier()

    def _wb(k, _):
        cid = s + _NS * k
        @pl.when(cid < _WB_CHUNKS)
        def _():
            pltpu.sync_copy(
                acc.at[pl.ds(cid * _WB_ROWS, _WB_ROWS)],
                out.at[pl.ds(hoff + cid * _WB_ROWS, _WB_ROWS)])
        return _
    lax.fori_loop(0, (_WB_CHUNKS + _NS - 1) // _NS, _wb, None)


_prop = functools.partial(
    pl.kernel,
    out_type=jax.ShapeDtypeStruct((_NC * _N, _DH), jnp.float32),
    compiler_params=pltpu.CompilerParams(use_tc_tiling_on_sc=False),
    mesh=plsc.VectorSubcoreMesh(core_axis_name="c", subcore_axis_name="s",
                                num_cores=_NC, num_subcores=_NS),
    scratch_types=(
        [
            pltpu.VMEM((_CHUNK,), jnp.int32),
            pltpu.VMEM((_CHUNK,), jnp.float32),
            pltpu.VMEM((_NSTR, _CSUB), jnp.int32),
            pltpu.VMEM((_CHUNK, _DH), jnp.float32),
        ] * 2
        + [pltpu.VMEM_SHARED((_N, _DH), jnp.float32)]
        + [pltpu.SemaphoreType.DMA] * 6
    ),
)(_prop_body)


_GB = 64
_GCHUNKS = 3 * _B // _GB


def _gather_body(e0, e1, e2, e3, items, pos, neg, g0, g1, g2, g3,
                 idxv, rowbuf, sem):
    c = lax.axis_index("c")
    s = lax.axis_index("s")
    w = s * _NC + c

    def _chunk(k, _):
        cid = w + _NC * _NS * k
        a = cid // (_B // _GB)
        q = cid % (_B // _GB)

        @pl.when(a == 0)
        def _():
            pltpu.sync_copy(items.at[pl.ds(q * _GB, _GB)], idxv)
        @pl.when(a == 1)
        def _():
            pltpu.sync_copy(pos.at[pl.ds(q * _GB, _GB)], idxv)
        @pl.when(a == 2)
        def _():
            pltpu.sync_copy(neg.at[pl.ds(q * _GB, _GB)], idxv)

        off = jnp.where(a == 0, 0, _NUM_ITEMS).astype(jnp.int32)
        for g in range(_GB // _L):
            idxv[pl.ds(g * _L, _L)] = idxv[pl.ds(g * _L, _L)] + off

        for h in range(_NC):
            if h:
                for g in range(_GB // _L):
                    idxv[pl.ds(g * _L, _L)] = idxv[pl.ds(g * _L, _L)] + _N
            for tbl, outt in ((e0, g0), (e1, g1), (e2, g2), (e3, g3)):
                pltpu.async_copy(tbl.at[idxv], rowbuf, sem).wait()
                pltpu.sync_copy(
                    rowbuf, outt.at[pl.ds(h * 3 * _B + cid * _GB, _GB)])
        return _
    lax.fori_loop(0, _GCHUNKS // (_NC * _NS), _chunk, None)


_gather = functools.partial(
    pl.kernel,
    out_type=(jax.ShapeDtypeStruct((_NC * 3 * _B, _DH), jnp.float32),) * 4,
    compiler_params=pltpu.CompilerParams(use_tc_tiling_on_sc=False),
    mesh=plsc.VectorSubcoreMesh(core_axis_name="c", subcore_axis_name="s",
                                num_cores=_NC, num_subcores=_NS),
    scratch_types=[
        pltpu.VMEM((_GB,), jnp.int32),
        pltpu.VMEM((_GB, _DH), jnp.float32),
        pltpu.SemaphoreType.DMA,
    ],
)(_gather_body)


def _loss_body(g0, g1, g2, g3, loss_ref, reg_ref):
    light = (g0[...] + g1[...] + g2[...] + g3[...]) * 0.25
    ps = jnp.zeros((_B,), jnp.float32)
    ns = jnp.zeros((_B,), jnp.float32)
    for h in range(_NC):
        o = h * 3 * _B
        items_emb = light[o:o + _B]
        pos_emb = light[o + _B:o + 2 * _B]
        neg_emb = light[o + 2 * _B:o + 3 * _B]
        ps = ps + jnp.sum(items_emb * pos_emb, axis=1)
        ns = ns + jnp.sum(items_emb * neg_emb, axis=1)
    loss_ref[0] = jnp.mean(jax.nn.softplus(ns - ps))
    reg_ref[0] = 0.5 * jnp.sum(g0[...] ** 2) / float(_B)


def _loss_stage(g0, g1, g2, g3):
    loss, reg = pl.pallas_call(
        _loss_body,
        out_shape=(
            jax.ShapeDtypeStruct((1,), jnp.float32),
            jax.ShapeDtypeStruct((1,), jnp.float32),
        ),
        in_specs=[pl.BlockSpec(memory_space=pltpu.VMEM)] * 4,
        out_specs=(
            pl.BlockSpec(memory_space=pltpu.SMEM),
            pl.BlockSpec(memory_space=pltpu.SMEM),
        ),
    )(g0, g1, g2, g3)
    return loss[0], reg[0]


def kernel(item_table, user_table, edge_vals, edge_index, items, pos, neg):
    e0 = jnp.concatenate([
        item_table[:, :_DH], user_table[:, :_DH],
        item_table[:, _DH:], user_table[:, _DH:],
    ], axis=0)
    row1 = edge_index[0]
    col1 = edge_index[1]

    e1 = _prop(e0, col1, row1, edge_vals)
    e2 = _prop(e1, col1, row1, edge_vals)
    e3 = _prop(e2, col1, row1, edge_vals)

    g0, g1, g2, g3 = _gather(e0, e1, e2, e3, items, pos, neg)
    loss, reg = _loss_stage(g0, g1, g2, g3)
    return (loss, reg)


# speedup vs baseline: 1.5573x; 1.0131x over previous
import functools

import jax
import jax.numpy as jnp
from jax import lax
from jax.experimental import pallas as pl
from jax.experimental.pallas import tpu as pltpu
from jax.experimental.pallas import tpu_sc as plsc

_NUM_ITEMS = 20000
_NUM_USERS = 30000
_N = _NUM_ITEMS + _NUM_USERS
_E = 800000
_D = 64
_DH = _D // 2
_NL = 3
_B = 2048

_NC = 2
_NS = 16
_L = 16

_CSUB = 128
_NSTR = 2
_CHUNK = _CSUB * _NSTR
_NCHUNKS = _E // _CHUNK
_NK = _NCHUNKS // _NS
_NLEFT = _NCHUNKS - _NK * _NS

_ZROWS = 125
_ZCHUNKS = _N // _ZROWS
_WB_ROWS = 200
_WB_CHUNKS = _N // _WB_ROWS


def _prop_body(emb, col1, row1, vals1, out,
               colv0, valsv0, sidx0, rows0,
               colv1, valsv1, sidx1, rows1,
               acc, gsem0, gsem1, ssem0, ssem1, isem0, isem1):
    c = lax.axis_index("c")
    s = lax.axis_index("s")
    hoff = c * _N

    colv = (colv0, colv1)
    valsv = (valsv0, valsv1)
    sidx = (sidx0, sidx1)
    rows = (rows0, rows1)
    gsem = (gsem0, gsem1)
    ssem = (ssem0, ssem1)
    isem = (isem0, isem1)

    def _zrow(r, _):
        for j in range(_DH // _L):
            rows0[r, pl.ds(j * _L, _L)] = jnp.zeros((_L,), jnp.float32)
        return _
    lax.fori_loop(0, _ZROWS, _zrow, None)

    def _zacc(k, _):
        cid = s + _NS * k
        pltpu.async_copy(rows0.at[pl.ds(0, _ZROWS)],
                         acc.at[pl.ds(cid * _ZROWS, _ZROWS)], gsem0)
        return _
    lax.fori_loop(0, _ZCHUNKS // _NS, _zacc, None)

    def _zacc_wait(k, _):
        cid = s + _NS * k
        pltpu.make_async_copy(rows0.at[pl.ds(0, _ZROWS)],
                              acc.at[pl.ds(cid * _ZROWS, _ZROWS)],
                              gsem0).wait()
        return _
    lax.fori_loop(0, _ZCHUNKS // _NS, _zacc_wait, None)
    plsc.subcore_barrier()

    def _ebase(k):
        return (k * _NS + s) * _CHUNK

    def _issue_idx(k, p):
        eb = _ebase(k)
        pltpu.async_copy(col1.at[pl.ds(eb, _CHUNK)], colv[p], isem[p])
        for j in range(_NSTR):
            pltpu.async_copy(row1.at[pl.ds(eb + j * _CSUB, _CSUB)],
                             sidx[p].at[j], isem[p])
        pltpu.async_copy(vals1.at[pl.ds(eb, _CHUNK)], valsv[p], isem[p])

    def _wait_idx(k, p):
        eb = _ebase(k)
        pltpu.make_async_copy(col1.at[pl.ds(eb, _CHUNK)], colv[p],
                              isem[p]).wait()
        for j in range(_NSTR):
            pltpu.make_async_copy(row1.at[pl.ds(eb + j * _CSUB, _CSUB)],
                                  sidx[p].at[j], isem[p]).wait()
        pltpu.make_async_copy(vals1.at[pl.ds(eb, _CHUNK)], valsv[p],
                              isem[p]).wait()
        @plsc.parallel_loop(0, _CHUNK // _L, unroll=2)
        def _adj(g):
            o = g * _L
            colv[p][pl.ds(o, _L)] = colv[p][pl.ds(o, _L)] + hoff

    def _issue_gather(p):
        for j in range(_NSTR):
            pltpu.async_copy(emb.at[colv[p].at[pl.ds(j * _CSUB, _CSUB)]],
                             rows[p].at[pl.ds(j * _CSUB, _CSUB)], gsem[p])

    def _wait_gather(p):
        for j in range(_NSTR):
            pltpu.make_async_copy(
                emb.at[colv[p].at[pl.ds(j * _CSUB, _CSUB)]],
                rows[p].at[pl.ds(j * _CSUB, _CSUB)], gsem[p]).wait()

    def _issue_scatter(p):
        for j in range(_NSTR):
            pltpu.async_copy(rows[p].at[pl.ds(j * _CSUB, _CSUB)],
                             acc.at[sidx[p].at[j]], ssem[p], add=True)

    def _wait_scatter(p):
        for j in range(_NSTR):
            pltpu.make_async_copy(rows[p].at[pl.ds(j * _CSUB, _CSUB)],
                                  acc.at[sidx[p].at[j]], ssem[p]).wait()

    def _compute(p):
        @plsc.parallel_loop(0, _CHUNK // _L, unroll=2)
        def _group(g):
            o = g * _L
            v16 = valsv[p][pl.ds(o, _L)]
            dn = lax.GatherDimensionNumbers(
                offset_dims=(), collapsed_slice_dims=(0,),
                start_index_map=(0,))
            for l in range(_L):
                idx = jnp.full((_L, 1), l, jnp.int32)
                splat = lax.gather(
                    v16, idx, dn, slice_sizes=(1,),
                    mode=lax.GatherScatterMode.PROMISE_IN_BOUNDS)
                for q in range(_DH // _L):
                    seg = rows[p][o + l, pl.ds(q * _L, _L)]
                    rows[p][o + l, pl.ds(q * _L, _L)] = seg * splat

    def _sub(k, p, first, last):
        _wait_gather(p)
        if not first:
            _wait_scatter(1 - p)
        if not last:
            _wait_idx(k + 1, 1 - p)
            _issue_gather(1 - p)
        _compute(p)
        _issue_scatter(p)
        if not last:
            @pl.when(k + 2 < _NK)
            def _():
                _issue_idx(k + 2, p)

    pltpu.sync_copy(col1.at[pl.ds(_ebase(0), _CHUNK)], colv[0])
    for j in range(_NSTR):
        pltpu.sync_copy(row1.at[pl.ds(_ebase(0) + j * _CSUB, _CSUB)],
                        sidx[0].at[j])
    pltpu.sync_copy(vals1.at[pl.ds(_ebase(0), _CHUNK)], valsv[0])

    @plsc.parallel_loop(0, _CHUNK // _L, unroll=2)
    def _adj0(g):
        o = g * _L
        colv[0][pl.ds(o, _L)] = colv[0][pl.ds(o, _L)] + hoff

    _issue_gather(0)
    _issue_idx(1, 1)

    def _dbody(t, _):
        k = 2 * t + 1
        _sub(k, 1, False, False)
        _sub(k + 1, 0, False, False)
        return _

    _sub(0, 0, True, False)
    lax.fori_loop(0, (_NK - 3) // 2, _dbody, None)
    _sub(_NK - 2, 1, False, False)
    _sub(_NK - 1, 0, False, True)
    _wait_scatter(0)

    @pl.when(s < _NLEFT)
    def _():
        eb = (_NK * _NS + s) * _CHUNK
        pltpu.sync_copy(col1.at[pl.ds(eb, _CHUNK)], colv[0])
        for j in range(_NSTR):
            pltpu.sync_copy(row1.at[pl.ds(eb + j * _CSUB, _CSUB)],
                            sidx[0].at[j])
        pltpu.sync_copy(vals1.at[pl.ds(eb, _CHUNK)], valsv[0])

        @plsc.parallel_loop(0, _CHUNK // _L, unroll=2)
        def _adjl(g):
            o = g * _L
            colv[0][pl.ds(o, _L)] = colv[0][pl.ds(o, _L)] + hoff

        _issue_gather(0)
        _wait_gather(0)
        _compute(0)
        _issue_scatter(0)
        _wait_scatter(0)

    plsc.subcore_barrier()

    def _wb(k, _):
        cid = s + _NS * k
        @pl.when(cid < _WB_CHUNKS)
        def _():
            pltpu.async_copy(
                acc.at[pl.ds(cid * _WB_ROWS, _WB_ROWS)],
                out.at[pl.ds(hoff + cid * _WB_ROWS, _WB_ROWS)], gsem0)
        return _
    lax.fori_loop(0, (_WB_CHUNKS + _NS - 1) // _NS, _wb, None)

    def _wb_wait(k, _):
        cid = s + _NS * k
        @pl.when(cid < _WB_CHUNKS)
        def _():
            pltpu.make_async_copy(
                acc.at[pl.ds(cid * _WB_ROWS, _WB_ROWS)],
                out.at[pl.ds(hoff + cid * _WB_ROWS, _WB_ROWS)],
                gsem0).wait()
        return _
    lax.fori_loop(0, (_WB_CHUNKS + _NS - 1) // _NS, _wb_wait, None)


_prop = functools.partial(
    pl.kernel,
    out_type=jax.ShapeDtypeStruct((_NC * _N, _DH), jnp.float32),
    compiler_params=pltpu.CompilerParams(use_tc_tiling_on_sc=False),
    mesh=plsc.VectorSubcoreMesh(core_axis_name="c", subcore_axis_name="s",
                                num_cores=_NC, num_subcores=_NS),
    scratch_types=(
        [
            pltpu.VMEM((_CHUNK,), jnp.int32),
            pltpu.VMEM((_CHUNK,), jnp.float32),
            pltpu.VMEM((_NSTR, _CSUB), jnp.int32),
            pltpu.VMEM((_CHUNK, _DH), jnp.float32),
        ] * 2
        + [pltpu.VMEM_SHARED((_N, _DH), jnp.float32)]
        + [pltpu.SemaphoreType.DMA] * 6
    ),
)(_prop_body)


_GB = 64
_GCHUNKS = 3 * _B // _GB


def _gather_body(e0, e1, e2, e3, items, pos, neg, g0, g1, g2, g3,
                 idxv, rowbuf, sem):
    c = lax.axis_index("c")
    s = lax.axis_index("s")
    w = s * _NC + c

    def _chunk(k, _):
        cid = w + _NC * _NS * k
        a = cid // (_B // _GB)
        q = cid % (_B // _GB)

        @pl.when(a == 0)
        def _():
            pltpu.sync_copy(items.at[pl.ds(q * _GB, _GB)], idxv)
        @pl.when(a == 1)
        def _():
            pltpu.sync_copy(pos.at[pl.ds(q * _GB, _GB)], idxv)
        @pl.when(a == 2)
        def _():
            pltpu.sync_copy(neg.at[pl.ds(q * _GB, _GB)], idxv)

        off = jnp.where(a == 0, 0, _NUM_ITEMS).astype(jnp.int32)
        for g in range(_GB // _L):
            idxv[pl.ds(g * _L, _L)] = idxv[pl.ds(g * _L, _L)] + off

        for h in range(_NC):
            if h:
                for g in range(_GB // _L):
                    idxv[pl.ds(g * _L, _L)] = idxv[pl.ds(g * _L, _L)] + _N
            for tbl, outt in ((e0, g0), (e1, g1), (e2, g2), (e3, g3)):
                pltpu.async_copy(tbl.at[idxv], rowbuf, sem).wait()
                pltpu.sync_copy(
                    rowbuf, outt.at[pl.ds(h * 3 * _B + cid * _GB, _GB)])
        return _
    lax.fori_loop(0, _GCHUNKS // (_NC * _NS), _chunk, None)


_gather = functools.partial(
    pl.kernel,
    out_type=(jax.ShapeDtypeStruct((_NC * 3 * _B, _DH), jnp.float32),) * 4,
    compiler_params=pltpu.CompilerParams(use_tc_tiling_on_sc=False),
    mesh=plsc.VectorSubcoreMesh(core_axis_name="c", subcore_axis_name="s",
                                num_cores=_NC, num_subcores=_NS),
    scratch_types=[
        pltpu.VMEM((_GB,), jnp.int32),
        pltpu.VMEM((_GB, _DH), jnp.float32),
        pltpu.SemaphoreType.DMA,
    ],
)(_gather_body)


def _loss_body(g0, g1, g2, g3, loss_ref, reg_ref):
    light = (g0[...] + g1[...] + g2[...] + g3[...]) * 0.25
    ps = jnp.zeros((_B,), jnp.float32)
    ns = jnp.zeros((_B,), jnp.float32)
    for h in range(_NC):
        o = h * 3 * _B
        items_emb = light[o:o + _B]
        pos_emb = light[o + _B:o + 2 * _B]
        neg_emb = light[o + 2 * _B:o + 3 * _B]
        ps = ps + jnp.sum(items_emb * pos_emb, axis=1)
        ns = ns + jnp.sum(items_emb * neg_emb, axis=1)
    loss_ref[0] = jnp.mean(jax.nn.softplus(ns - ps))
    reg_ref[0] = 0.5 * jnp.sum(g0[...] ** 2) / float(_B)


def _loss_stage(g0, g1, g2, g3):
    loss, reg = pl.pallas_call(
        _loss_body,
        out_shape=(
            jax.ShapeDtypeStruct((1,), jnp.float32),
            jax.ShapeDtypeStruct((1,), jnp.float32),
        ),
        in_specs=[pl.BlockSpec(memory_space=pltpu.VMEM)] * 4,
        out_specs=(
            pl.BlockSpec(memory_space=pltpu.SMEM),
            pl.BlockSpec(memory_space=pltpu.SMEM),
        ),
    )(g0, g1, g2, g3)
    return loss[0], reg[0]


def kernel(item_table, user_table, edge_vals, edge_index, items, pos, neg):
    e0 = jnp.concatenate([
        item_table[:, :_DH], user_table[:, :_DH],
        item_table[:, _DH:], user_table[:, _DH:],
    ], axis=0)
    row1 = edge_index[0]
    col1 = edge_index[1]

    e1 = _prop(e0, col1, row1, edge_vals)
    e2 = _prop(e1, col1, row1, edge_vals)
    e3 = _prop(e2, col1, row1, edge_vals)

    g0, g1, g2, g3 = _gather(e0, e1, e2, e3, items, pos, neg)
    loss, reg = _loss_stage(g0, g1, g2, g3)
    return (loss, reg)


# speedup vs baseline: 1.5777x; 1.0131x over previous
import functools

import jax
import jax.numpy as jnp
from jax import lax
from jax.experimental import pallas as pl
from jax.experimental.pallas import tpu as pltpu
from jax.experimental.pallas import tpu_sc as plsc

_NUM_ITEMS = 20000
_NUM_USERS = 30000
_N = _NUM_ITEMS + _NUM_USERS
_E = 800000
_D = 64
_DH = _D // 2
_NL = 3
_B = 2048

_NC = 2
_NS = 16
_L = 16

_CSUB = 128
_NSTR = 2
_CHUNK = _CSUB * _NSTR
_NCHUNKS = _E // _CHUNK
_NK = _NCHUNKS // _NS
_NLEFT = _NCHUNKS - _NK * _NS

_ZROWS = 125
_ZCHUNKS = _N // _ZROWS
_WB_ROWS = 200
_WB_CHUNKS = _N // _WB_ROWS


def _mega_body(e0, col1, row1, vals1, items, pos, neg,
               o1, o2, o3, g0, g1, g2, g3,
               colv0, valsv0, sidx0, rows0,
               colv1, valsv1, sidx1, rows1,
               acc, gsem0, gsem1, ssem0, ssem1, isem0, isem1):
    c = lax.axis_index("c")
    s = lax.axis_index("s")
    hoff = c * _N

    for emb, out in ((e0, o1), (o1, o2), (o2, o3)):
        _layer(emb, out, col1, row1, vals1, c, s, hoff,
               (colv0, colv1), (valsv0, valsv1), (sidx0, sidx1),
               (rows0, rows1), acc,
               (gsem0, gsem1), (ssem0, ssem1), (isem0, isem1))

    def _gchunk(k, _):
        cid = s + _NS * k
        a = cid // (_B // _GB)
        q = cid % (_B // _GB)
        idxv = colv0.at[pl.ds(0, _GB)]
        rowbuf = rows0.at[pl.ds(0, _GB)]

        @pl.when(a == 0)
        def _():
            pltpu.sync_copy(items.at[pl.ds(q * _GB, _GB)], idxv)
        @pl.when(a == 1)
        def _():
            pltpu.sync_copy(pos.at[pl.ds(q * _GB, _GB)], idxv)
        @pl.when(a == 2)
        def _():
            pltpu.sync_copy(neg.at[pl.ds(q * _GB, _GB)], idxv)

        off = (jnp.where(a == 0, 0, _NUM_ITEMS) + hoff).astype(jnp.int32)
        for g in range(_GB // _L):
            idxv[pl.ds(g * _L, _L)] = idxv[pl.ds(g * _L, _L)] + off

        for tbl, outt in ((e0, g0), (o1, g1), (o2, g2), (o3, g3)):
            pltpu.async_copy(tbl.at[idxv], rowbuf, gsem0).wait()
            pltpu.sync_copy(
                rowbuf, outt.at[pl.ds(c * 3 * _B + cid * _GB, _GB)])
        return _
    lax.fori_loop(0, _GCHUNKS // _NS, _gchunk, None)


def _layer(emb, out, col1, row1, vals1, c, s, hoff,
           colv, valsv, sidx, rows, acc, gsem, ssem, isem):
    rows0 = rows[0]
    gsem0 = gsem[0]

    def _zrow(r, _):
        for j in range(_DH // _L):
            rows0[r, pl.ds(j * _L, _L)] = jnp.zeros((_L,), jnp.float32)
        return _
    lax.fori_loop(0, _ZROWS, _zrow, None)

    def _zacc(k, _):
        cid = s + _NS * k
        pltpu.async_copy(rows0.at[pl.ds(0, _ZROWS)],
                         acc.at[pl.ds(cid * _ZROWS, _ZROWS)], gsem0)
        return _
    lax.fori_loop(0, _ZCHUNKS // _NS, _zacc, None)

    def _zacc_wait(k, _):
        cid = s + _NS * k
        pltpu.make_async_copy(rows0.at[pl.ds(0, _ZROWS)],
                              acc.at[pl.ds(cid * _ZROWS, _ZROWS)],
                              gsem0).wait()
        return _
    lax.fori_loop(0, _ZCHUNKS // _NS, _zacc_wait, None)
    plsc.subcore_barrier()

    def _ebase(k):
        return (k * _NS + s) * _CHUNK

    def _issue_idx(k, p):
        eb = _ebase(k)
        pltpu.async_copy(col1.at[pl.ds(eb, _CHUNK)], colv[p], isem[p])
        for j in range(_NSTR):
            pltpu.async_copy(row1.at[pl.ds(eb + j * _CSUB, _CSUB)],
                             sidx[p].at[j], isem[p])
        pltpu.async_copy(vals1.at[pl.ds(eb, _CHUNK)], valsv[p], isem[p])

    def _wait_idx(k, p):
        eb = _ebase(k)
        pltpu.make_async_copy(col1.at[pl.ds(eb, _CHUNK)], colv[p],
                              isem[p]).wait()
        for j in range(_NSTR):
            pltpu.make_async_copy(row1.at[pl.ds(eb + j * _CSUB, _CSUB)],
                                  sidx[p].at[j], isem[p]).wait()
        pltpu.make_async_copy(vals1.at[pl.ds(eb, _CHUNK)], valsv[p],
                              isem[p]).wait()
        @plsc.parallel_loop(0, _CHUNK // _L, unroll=2)
        def _adj(g):
            o = g * _L
            colv[p][pl.ds(o, _L)] = colv[p][pl.ds(o, _L)] + hoff

    def _issue_gather(p):
        for j in range(_NSTR):
            pltpu.async_copy(emb.at[colv[p].at[pl.ds(j * _CSUB, _CSUB)]],
                             rows[p].at[pl.ds(j * _CSUB, _CSUB)], gsem[p])

    def _wait_gather(p):
        for j in range(_NSTR):
            pltpu.make_async_copy(
                emb.at[colv[p].at[pl.ds(j * _CSUB, _CSUB)]],
                rows[p].at[pl.ds(j * _CSUB, _CSUB)], gsem[p]).wait()

    def _issue_scatter(p):
        for j in range(_NSTR):
            pltpu.async_copy(rows[p].at[pl.ds(j * _CSUB, _CSUB)],
                             acc.at[sidx[p].at[j]], ssem[p], add=True)

    def _wait_scatter(p):
        for j in range(_NSTR):
            pltpu.make_async_copy(rows[p].at[pl.ds(j * _CSUB, _CSUB)],
                                  acc.at[sidx[p].at[j]], ssem[p]).wait()

    def _compute(p):
        @plsc.parallel_loop(0, _CHUNK // _L, unroll=2)
        def _group(g):
            o = g * _L
            v16 = valsv[p][pl.ds(o, _L)]
            dn = lax.GatherDimensionNumbers(
                offset_dims=(), collapsed_slice_dims=(0,),
                start_index_map=(0,))
            for l in range(_L):
                idx = jnp.full((_L, 1), l, jnp.int32)
                splat = lax.gather(
                    v16, idx, dn, slice_sizes=(1,),
                    mode=lax.GatherScatterMode.PROMISE_IN_BOUNDS)
                for q in range(_DH // _L):
                    seg = rows[p][o + l, pl.ds(q * _L, _L)]
                    rows[p][o + l, pl.ds(q * _L, _L)] = seg * splat

    def _sub(k, p, first, last):
        _wait_gather(p)
        if not first:
            _wait_scatter(1 - p)
        if not last:
            _wait_idx(k + 1, 1 - p)
            _issue_gather(1 - p)
        _compute(p)
        _issue_scatter(p)
        if not last:
            @pl.when(k + 2 < _NK)
            def _():
                _issue_idx(k + 2, p)

    pltpu.sync_copy(col1.at[pl.ds(_ebase(0), _CHUNK)], colv[0])
    for j in range(_NSTR):
        pltpu.sync_copy(row1.at[pl.ds(_ebase(0) + j * _CSUB, _CSUB)],
                        sidx[0].at[j])
    pltpu.sync_copy(vals1.at[pl.ds(_ebase(0), _CHUNK)], valsv[0])

    @plsc.parallel_loop(0, _CHUNK // _L, unroll=2)
    def _adj0(g):
        o = g * _L
        colv[0][pl.ds(o, _L)] = colv[0][pl.ds(o, _L)] + hoff

    _issue_gather(0)
    _issue_idx(1, 1)

    def _dbody(t, _):
        k = 2 * t + 1
        _sub(k, 1, False, False)
        _sub(k + 1, 0, False, False)
        return _

    _sub(0, 0, True, False)
    lax.fori_loop(0, (_NK - 3) // 2, _dbody, None)
    _sub(_NK - 2, 1, False, False)
    _sub(_NK - 1, 0, False, True)
    _wait_scatter(0)

    @pl.when(s < _NLEFT)
    def _():
        eb = (_NK * _NS + s) * _CHUNK
        pltpu.sync_copy(col1.at[pl.ds(eb, _CHUNK)], colv[0])
        for j in range(_NSTR):
            pltpu.sync_copy(row1.at[pl.ds(eb + j * _CSUB, _CSUB)],
                            sidx[0].at[j])
        pltpu.sync_copy(vals1.at[pl.ds(eb, _CHUNK)], valsv[0])

        @plsc.parallel_loop(0, _CHUNK // _L, unroll=2)
        def _adjl(g):
            o = g * _L
            colv[0][pl.ds(o, _L)] = colv[0][pl.ds(o, _L)] + hoff

        _issue_gather(0)
        _wait_gather(0)
        _compute(0)
        _issue_scatter(0)
        _wait_scatter(0)

    plsc.subcore_barrier()

    def _wb(k, _):
        cid = s + _NS * k
        @pl.when(cid < _WB_CHUNKS)
        def _():
            pltpu.async_copy(
                acc.at[pl.ds(cid * _WB_ROWS, _WB_ROWS)],
                out.at[pl.ds(hoff + cid * _WB_ROWS, _WB_ROWS)], gsem0)
        return _
    lax.fori_loop(0, (_WB_CHUNKS + _NS - 1) // _NS, _wb, None)

    def _wb_wait(k, _):
        cid = s + _NS * k
        @pl.when(cid < _WB_CHUNKS)
        def _():
            pltpu.make_async_copy(
                acc.at[pl.ds(cid * _WB_ROWS, _WB_ROWS)],
                out.at[pl.ds(hoff + cid * _WB_ROWS, _WB_ROWS)],
                gsem0).wait()
        return _
    lax.fori_loop(0, (_WB_CHUNKS + _NS - 1) // _NS, _wb_wait, None)
    plsc.subcore_barrier()


_GB = 64
_GCHUNKS = 3 * _B // _GB

_mega = functools.partial(
    pl.kernel,
    out_type=(
        (jax.ShapeDtypeStruct((_NC * _N, _DH), jnp.float32),) * 3
        + (jax.ShapeDtypeStruct((_NC * 3 * _B, _DH), jnp.float32),) * 4
    ),
    compiler_params=pltpu.CompilerParams(use_tc_tiling_on_sc=False),
    mesh=plsc.VectorSubcoreMesh(core_axis_name="c", subcore_axis_name="s",
                                num_cores=_NC, num_subcores=_NS),
    scratch_types=(
        [
            pltpu.VMEM((_CHUNK,), jnp.int32),
            pltpu.VMEM((_CHUNK,), jnp.float32),
            pltpu.VMEM((_NSTR, _CSUB), jnp.int32),
            pltpu.VMEM((_CHUNK, _DH), jnp.float32),
        ] * 2
        + [pltpu.VMEM_SHARED((_N, _DH), jnp.float32)]
        + [pltpu.SemaphoreType.DMA] * 6
    ),
)(_mega_body)


def _loss_body(g0, g1, g2, g3, loss_ref, reg_ref):
    light = (g0[...] + g1[...] + g2[...] + g3[...]) * 0.25
    ps = jnp.zeros((_B,), jnp.float32)
    ns = jnp.zeros((_B,), jnp.float32)
    for h in range(_NC):
        o = h * 3 * _B
        items_emb = light[o:o + _B]
        pos_emb = light[o + _B:o + 2 * _B]
        neg_emb = light[o + 2 * _B:o + 3 * _B]
        ps = ps + jnp.sum(items_emb * pos_emb, axis=1)
        ns = ns + jnp.sum(items_emb * neg_emb, axis=1)
    loss_ref[0] = jnp.mean(jax.nn.softplus(ns - ps))
    reg_ref[0] = 0.5 * jnp.sum(g0[...] ** 2) / float(_B)


def _loss_stage(g0, g1, g2, g3):
    loss, reg = pl.pallas_call(
        _loss_body,
        out_shape=(
            jax.ShapeDtypeStruct((1,), jnp.float32),
            jax.ShapeDtypeStruct((1,), jnp.float32),
        ),
        in_specs=[pl.BlockSpec(memory_space=pltpu.VMEM)] * 4,
        out_specs=(
            pl.BlockSpec(memory_space=pltpu.SMEM),
            pl.BlockSpec(memory_space=pltpu.SMEM),
        ),
    )(g0, g1, g2, g3)
    return loss[0], reg[0]


def kernel(item_table, user_table, edge_vals, edge_index, items, pos, neg):
    e0 = jnp.concatenate([
        item_table[:, :_DH], user_table[:, :_DH],
        item_table[:, _DH:], user_table[:, _DH:],
    ], axis=0)
    row1 = edge_index[0]
    col1 = edge_index[1]

    _o1, _o2, _o3, g0, g1, g2, g3 = _mega(
        e0, col1, row1, edge_vals, items, pos, neg)
    loss, reg = _loss_stage(g0, g1, g2, g3)
    return (loss, reg)


# speedup vs baseline: 1.6578x; 1.0508x over previous
import functools

import jax
import jax.numpy as jnp
from jax import lax
from jax.experimental import pallas as pl
from jax.experimental.pallas import tpu as pltpu
from jax.experimental.pallas import tpu_sc as plsc

_NUM_ITEMS = 20000
_NUM_USERS = 30000
_N = _NUM_ITEMS + _NUM_USERS
_E = 800000
_D = 64
_DH = _D // 2
_NL = 3
_B = 2048

_NC = 2
_NS = 16
_L = 16

_CSUB = 128
_NSTR = 2
_CHUNK = _CSUB * _NSTR
_NCHUNKS = _E // _CHUNK
_NK = _NCHUNKS // _NS
_NLEFT = _NCHUNKS - _NK * _NS

_ZROWS = 125
_ZCHUNKS = _N // _ZROWS
_WB_ROWS = 200
_WB_CHUNKS = _N // _WB_ROWS


def _mega_body(e0, col1, row1, vals1, items, pos, neg,
               o1, o2, o3, g0, g1, g2, g3,
               colv0, valsv0, sidx0, rows0,
               colv1, valsv1, sidx1, rows1,
               acc, gsem0, gsem1, ssem0, ssem1, isem0, isem1):
    c = lax.axis_index("c")
    s = lax.axis_index("s")
    hoff = c * _N

    for emb, out in ((e0, o1), (o1, o2), (o2, o3)):
        _layer(emb, out, col1, row1, vals1, c, s, hoff,
               (colv0, colv1), (valsv0, valsv1), (sidx0, sidx1),
               (rows0, rows1), acc,
               (gsem0, gsem1), (ssem0, ssem1), (isem0, isem1))

    def _gchunk(k, _):
        cid = s + _NS * k
        a = cid // (_B // _GB)
        q = cid % (_B // _GB)
        idxv = colv0.at[pl.ds(0, _GB)]
        rowbuf = rows0.at[pl.ds(0, _GB)]

        @pl.when(a == 0)
        def _():
            pltpu.sync_copy(items.at[pl.ds(q * _GB, _GB)], idxv)
        @pl.when(a == 1)
        def _():
            pltpu.sync_copy(pos.at[pl.ds(q * _GB, _GB)], idxv)
        @pl.when(a == 2)
        def _():
            pltpu.sync_copy(neg.at[pl.ds(q * _GB, _GB)], idxv)

        off = (jnp.where(a == 0, 0, _NUM_ITEMS) + hoff).astype(jnp.int32)
        for g in range(_GB // _L):
            idxv[pl.ds(g * _L, _L)] = idxv[pl.ds(g * _L, _L)] + off

        for tbl, outt in ((e0, g0), (o1, g1), (o2, g2), (o3, g3)):
            pltpu.async_copy(tbl.at[idxv], rowbuf, gsem0).wait()
            pltpu.sync_copy(
                rowbuf, outt.at[pl.ds(c * 3 * _B + cid * _GB, _GB)])
        return _
    lax.fori_loop(0, _GCHUNKS // _NS, _gchunk, None)


def _layer(emb, out, col1, row1, vals1, c, s, hoff,
           colv, valsv, sidx, rows, acc, gsem, ssem, isem):
    rows0 = rows[0]
    gsem0 = gsem[0]

    def _zrow(r, _):
        for j in range(_DH // _L):
            rows0[r, pl.ds(j * _L, _L)] = jnp.zeros((_L,), jnp.float32)
        return _
    lax.fori_loop(0, _ZROWS, _zrow, None)

    def _zacc(k, _):
        cid = s + _NS * k
        pltpu.async_copy(rows0.at[pl.ds(0, _ZROWS)],
                         acc.at[pl.ds(cid * _ZROWS, _ZROWS)], gsem0)
        return _
    lax.fori_loop(0, _ZCHUNKS // _NS, _zacc, None)

    def _zacc_wait(k, _):
        cid = s + _NS * k
        pltpu.make_async_copy(rows0.at[pl.ds(0, _ZROWS)],
                              acc.at[pl.ds(cid * _ZROWS, _ZROWS)],
                              gsem0).wait()
        return _
    lax.fori_loop(0, _ZCHUNKS // _NS, _zacc_wait, None)
    plsc.subcore_barrier()

    def _ebase(k):
        return (k * _NS + s) * _CHUNK

    def _issue_idx(k, p):
        eb = _ebase(k)
        pltpu.async_copy(col1.at[pl.ds(eb, _CHUNK)], colv[p], isem[p])
        for j in range(_NSTR):
            pltpu.async_copy(row1.at[pl.ds(eb + j * _CSUB, _CSUB)],
                             sidx[p].at[j], isem[p])
        pltpu.async_copy(vals1.at[pl.ds(eb, _CHUNK)], valsv[p], isem[p])

    def _wait_idx(k, p):
        eb = _ebase(k)
        pltpu.make_async_copy(col1.at[pl.ds(eb, _CHUNK)], colv[p],
                              isem[p]).wait()
        for j in range(_NSTR):
            pltpu.make_async_copy(row1.at[pl.ds(eb + j * _CSUB, _CSUB)],
                                  sidx[p].at[j], isem[p]).wait()
        pltpu.make_async_copy(vals1.at[pl.ds(eb, _CHUNK)], valsv[p],
                              isem[p]).wait()
        @plsc.parallel_loop(0, _CHUNK // _L, unroll=2)
        def _adj(g):
            o = g * _L
            colv[p][pl.ds(o, _L)] = colv[p][pl.ds(o, _L)] + hoff

    def _issue_gather(p):
        for j in range(_NSTR):
            pltpu.async_copy(emb.at[colv[p].at[pl.ds(j * _CSUB, _CSUB)]],
                             rows[p].at[pl.ds(j * _CSUB, _CSUB)], gsem[p])

    def _wait_gather(p):
        for j in range(_NSTR):
            pltpu.make_async_copy(
                emb.at[colv[p].at[pl.ds(j * _CSUB, _CSUB)]],
                rows[p].at[pl.ds(j * _CSUB, _CSUB)], gsem[p]).wait()

    def _issue_scatter(p):
        for j in range(_NSTR):
            pltpu.async_copy(rows[p].at[pl.ds(j * _CSUB, _CSUB)],
                             acc.at[sidx[p].at[j]], ssem[p], add=True)

    def _wait_scatter(p):
        for j in range(_NSTR):
            pltpu.make_async_copy(rows[p].at[pl.ds(j * _CSUB, _CSUB)],
                                  acc.at[sidx[p].at[j]], ssem[p]).wait()

    def _compute(p):
        @plsc.parallel_loop(0, _CHUNK // _L, unroll=2)
        def _group(g):
            o = g * _L
            v16 = valsv[p][pl.ds(o, _L)]
            dn = lax.GatherDimensionNumbers(
                offset_dims=(), collapsed_slice_dims=(0,),
                start_index_map=(0,))
            for l in range(_L):
                idx = jnp.full((_L, 1), l, jnp.int32)
                splat = lax.gather(
                    v16, idx, dn, slice_sizes=(1,),
                    mode=lax.GatherScatterMode.PROMISE_IN_BOUNDS)
                for q in range(_DH // _L):
                    seg = rows[p][o + l, pl.ds(q * _L, _L)]
                    rows[p][o + l, pl.ds(q * _L, _L)] = seg * splat

    def _sub(k, p, first, last):
        _wait_gather(p)
        if not first:
            _wait_scatter(1 - p)
        if not last:
            _wait_idx(k + 1, 1 - p)
            _issue_gather(1 - p)
        _compute(p)
        _issue_scatter(p)
        if not last:
            @pl.when(k + 2 < _NK)
            def _():
                _issue_idx(k + 2, p)

    pltpu.sync_copy(col1.at[pl.ds(_ebase(0), _CHUNK)], colv[0])
    for j in range(_NSTR):
        pltpu.sync_copy(row1.at[pl.ds(_ebase(0) + j * _CSUB, _CSUB)],
                        sidx[0].at[j])
    pltpu.sync_copy(vals1.at[pl.ds(_ebase(0), _CHUNK)], valsv[0])

    @plsc.parallel_loop(0, _CHUNK // _L, unroll=2)
    def _adj0(g):
        o = g * _L
        colv[0][pl.ds(o, _L)] = colv[0][pl.ds(o, _L)] + hoff

    _issue_gather(0)
    _issue_idx(1, 1)

    def _dbody(t, _):
        k = 2 * t + 1
        _sub(k, 1, False, False)
        _sub(k + 1, 0, False, False)
        return _

    _sub(0, 0, True, False)
    lax.fori_loop(0, (_NK - 3) // 2, _dbody, None)
    _sub(_NK - 2, 1, False, False)
    _sub(_NK - 1, 0, False, True)
    _wait_scatter(0)

    @pl.when(s < _NLEFT)
    def _():
        eb = (_NK * _NS + s) * _CHUNK
        pltpu.sync_copy(col1.at[pl.ds(eb, _CHUNK)], colv[0])
        for j in range(_NSTR):
            pltpu.sync_copy(row1.at[pl.ds(eb + j * _CSUB, _CSUB)],
                            sidx[0].at[j])
        pltpu.sync_copy(vals1.at[pl.ds(eb, _CHUNK)], valsv[0])

        @plsc.parallel_loop(0, _CHUNK // _L, unroll=2)
        def _adjl(g):
            o = g * _L
            colv[0][pl.ds(o, _L)] = colv[0][pl.ds(o, _L)] + hoff

        _issue_gather(0)
        _wait_gather(0)
        _compute(0)
        _issue_scatter(0)
        _wait_scatter(0)

    plsc.subcore_barrier()

    def _wb(k, _):
        cid = s + _NS * k
        @pl.when(cid < _WB_CHUNKS)
        def _():
            pltpu.async_copy(
                acc.at[pl.ds(cid * _WB_ROWS, _WB_ROWS)],
                out.at[pl.ds(hoff + cid * _WB_ROWS, _WB_ROWS)], gsem0)
        return _
    lax.fori_loop(0, (_WB_CHUNKS + _NS - 1) // _NS, _wb, None)

    def _wb_wait(k, _):
        cid = s + _NS * k
        @pl.when(cid < _WB_CHUNKS)
        def _():
            pltpu.make_async_copy(
                acc.at[pl.ds(cid * _WB_ROWS, _WB_ROWS)],
                out.at[pl.ds(hoff + cid * _WB_ROWS, _WB_ROWS)],
                gsem0).wait()
        return _
    lax.fori_loop(0, (_WB_CHUNKS + _NS - 1) // _NS, _wb_wait, None)
    plsc.subcore_barrier()


_GB = 64
_GCHUNKS = 3 * _B // _GB

_mega = functools.partial(
    pl.kernel,
    out_type=(
        (jax.ShapeDtypeStruct((_NC * _N, _DH), jnp.float32),) * 3
        + (jax.ShapeDtypeStruct((_NC * 3 * _B, _DH), jnp.float32),) * 4
    ),
    compiler_params=pltpu.CompilerParams(use_tc_tiling_on_sc=False),
    mesh=plsc.VectorSubcoreMesh(core_axis_name="c", subcore_axis_name="s",
                                num_cores=_NC, num_subcores=_NS),
    scratch_types=(
        [
            pltpu.VMEM((_CHUNK,), jnp.int32),
            pltpu.VMEM((_CHUNK,), jnp.float32),
            pltpu.VMEM((_NSTR, _CSUB), jnp.int32),
            pltpu.VMEM((_CHUNK, _DH), jnp.float32),
        ] * 2
        + [pltpu.VMEM_SHARED((_N, _DH), jnp.float32)]
        + [pltpu.SemaphoreType.DMA] * 6
    ),
)(_mega_body)


def _loss_body(g0, g1, g2, g3, loss_ref, reg_ref):
    light = (g0[...] + g1[...] + g2[...] + g3[...]) * 0.25
    ps = jnp.zeros((_B,), jnp.float32)
    ns = jnp.zeros((_B,), jnp.float32)
    for h in range(_NC):
        o = h * 3 * _B
        items_emb = light[o:o + _B]
        pos_emb = light[o + _B:o + 2 * _B]
        neg_emb = light[o + 2 * _B:o + 3 * _B]
        ps = ps + jnp.sum(items_emb * pos_emb, axis=1)
        ns = ns + jnp.sum(items_emb * neg_emb, axis=1)
    loss_ref[0] = jnp.mean(jax.nn.softplus(ns - ps))
    reg_ref[0] = 0.5 * jnp.sum(g0[...] ** 2) / float(_B)


def _loss_stage(g0, g1, g2, g3):
    loss, reg = pl.pallas_call(
        _loss_body,
        out_shape=(
            jax.ShapeDtypeStruct((1,), jnp.float32),
            jax.ShapeDtypeStruct((1,), jnp.float32),
        ),
        in_specs=[pl.BlockSpec(memory_space=pltpu.VMEM)] * 4,
        out_specs=(
            pl.BlockSpec(memory_space=pltpu.SMEM),
            pl.BlockSpec(memory_space=pltpu.SMEM),
        ),
    )(g0, g1, g2, g3)
    return loss[0], reg[0]


def kernel(item_table, user_table, edge_vals, edge_index, items, pos, neg):
    cat = jnp.concatenate([item_table, user_table], axis=0)
    e0 = cat.reshape(_N, 2, _DH).transpose(1, 0, 2).reshape(_NC * _N, _DH)
    row1 = edge_index[0]
    col1 = edge_index[1]

    _o1, _o2, _o3, g0, g1, g2, g3 = _mega(
        e0, col1, row1, edge_vals, items, pos, neg)
    loss, reg = _loss_stage(g0, g1, g2, g3)
    return (loss, reg)
